# baseline (speedup 1.0000x reference)
import functools

import jax
import jax.numpy as jnp
from jax import lax
from jax.experimental import pallas as pl
from jax.experimental.pallas import tpu as pltpu
from jax.experimental.pallas import tpu_sc as plsc

_B = 16384
_L = 20
_D = 32

_info = plsc.get_sparse_core_info()
_NC = _info.num_cores
_NS = _info.num_subcores
_NW = _NC * _NS
_RW = _B // _NW
_NBUF = 5

_mesh = plsc.VectorSubcoreMesh(core_axis_name="c", subcore_axis_name="s")


@functools.partial(
    pl.kernel,
    mesh=_mesh,
    out_type=jax.ShapeDtypeStruct((_L, _B, _D), jnp.float32),
    scratch_types=[
        pltpu.VMEM((_L, _RW), jnp.int32),
        pltpu.VMEM((_NBUF, _RW, _D), jnp.float32),
        pltpu.SemaphoreType.DMA((_NBUF,)),
        pltpu.SemaphoreType.DMA((_NBUF,)),
    ],
    compiler_params=pltpu.CompilerParams(
        use_tc_tiling_on_sc=False, needs_layout_passes=False),
)
def _gather(idx_hbm, table_hbm, out_hbm, idx_v, bufs, gsems, ssems):
    wid = lax.axis_index("s") * _NC + lax.axis_index("c")
    row0 = wid * _RW

    for l in range(_L):
        pltpu.sync_copy(idx_hbm.at[l, pl.ds(row0, _RW)], idx_v.at[l])

    def start_gather(g):
        b = g % _NBUF
        return pltpu.async_copy(
            table_hbm.at[idx_v.at[g]], bufs.at[b], gsems.at[b])

    def start_store(g):
        b = g % _NBUF
        return pltpu.async_copy(
            bufs.at[b], out_hbm.at[g, pl.ds(row0, _RW)], ssems.at[b])

    gcopies = [None] * _L
    scopies = [None] * _L
    for g in range(min(_NBUF, _L)):
        gcopies[g] = start_gather(g)
    for g in range(_L):
        ng = g - 1 + _NBUF
        if g >= 1 and ng < _L:
            scopies[g - 1].wait()
            gcopies[ng] = start_gather(ng)
        gcopies[g].wait()
        scopies[g] = start_store(g)
    for g in range(max(_L - _NBUF, 0), _L):
        if scopies[g] is not None:
            scopies[g].wait()


def kernel(token_id, weight):
    out_lbd = _gather(token_id.T, weight)
    return jnp.transpose(out_lbd, (1, 0, 2))


# speedup vs baseline: 1.0149x; 1.0149x over previous
import functools

import jax
import jax.numpy as jnp
from jax import lax
from jax.experimental import pallas as pl
from jax.experimental.pallas import tpu as pltpu
from jax.experimental.pallas import tpu_sc as plsc

_B = 16384
_L = 20
_D = 32

_info = plsc.get_sparse_core_info()
_NC = _info.num_cores
_NS = _info.num_subcores
_NW = _NC * _NS
_RW = _B // _NW
_NBUF = 5

_mesh = plsc.VectorSubcoreMesh(core_axis_name="c", subcore_axis_name="s")


@functools.partial(
    pl.kernel,
    mesh=_mesh,
    out_type=jax.ShapeDtypeStruct((_L, _B, _D), jnp.float32),
    scratch_types=[
        pltpu.VMEM((_L, _RW), jnp.int32),
        pltpu.VMEM((_NBUF, _RW, _D), jnp.float32),
        pltpu.SemaphoreType.DMA((_NBUF,)),
        pltpu.SemaphoreType.DMA((_NBUF,)),
        pltpu.SemaphoreType.DMA,
    ],
    compiler_params=pltpu.CompilerParams(
        use_tc_tiling_on_sc=False, needs_layout_passes=False),
)
def _gather(idx_hbm, table_hbm, out_hbm, idx_v, bufs, gsems, ssems, isem):
    wid = lax.axis_index("s") * _NC + lax.axis_index("c")
    row0 = wid * _RW

    icopies = [
        pltpu.async_copy(idx_hbm.at[l, pl.ds(row0, _RW)], idx_v.at[l], isem)
        for l in range(_L)
    ]
    for c in icopies:
        c.wait()

    def start_gather(g):
        b = g % _NBUF
        return pltpu.async_copy(
            table_hbm.at[idx_v.at[g]], bufs.at[b], gsems.at[b])

    def start_store(g):
        b = g % _NBUF
        return pltpu.async_copy(
            bufs.at[b], out_hbm.at[g, pl.ds(row0, _RW)], ssems.at[b])

    gcopies = [None] * _L
    scopies = [None] * _L
    for g in range(min(_NBUF, _L)):
        gcopies[g] = start_gather(g)
    for g in range(_L):
        ng = g - 1 + _NBUF
        if g >= 1 and ng < _L:
            scopies[g - 1].wait()
            gcopies[ng] = start_gather(ng)
        gcopies[g].wait()
        scopies[g] = start_store(g)
    for g in range(max(_L - _NBUF, 0), _L):
        if scopies[g] is not None:
            scopies[g].wait()


def kernel(token_id, weight):
    out_lbd = _gather(token_id.T, weight)
    return jnp.transpose(out_lbd, (1, 0, 2))


# speedup vs baseline: 1.0178x; 1.0028x over previous
import functools

import jax
import jax.numpy as jnp
from jax import lax
from jax.experimental import pallas as pl
from jax.experimental.pallas import tpu as pltpu
from jax.experimental.pallas import tpu_sc as plsc

_B = 16384
_L = 20
_D = 32

_info = plsc.get_sparse_core_info()
_NC = _info.num_cores
_NS = _info.num_subcores
_NW = _NC * _NS
_RW = _B // _NW
_NBUF = 6

_mesh = plsc.VectorSubcoreMesh(core_axis_name="c", subcore_axis_name="s")


@functools.partial(
    pl.kernel,
    mesh=_mesh,
    out_type=jax.ShapeDtypeStruct((_L, _B, _D), jnp.float32),
    scratch_types=[
        pltpu.VMEM((_L, _RW), jnp.int32),
        pltpu.VMEM((_NBUF, _RW, _D), jnp.float32),
        pltpu.SemaphoreType.DMA((_NBUF,)),
        pltpu.SemaphoreType.DMA((_NBUF,)),
        pltpu.SemaphoreType.DMA,
    ],
    compiler_params=pltpu.CompilerParams(
        use_tc_tiling_on_sc=False, needs_layout_passes=False),
)
def _gather(idx_hbm, table_hbm, out_hbm, idx_v, bufs, gsems, ssems, isem):
    wid = lax.axis_index("s") * _NC + lax.axis_index("c")
    row0 = wid * _RW

    icopies = [
        pltpu.async_copy(idx_hbm.at[l, pl.ds(row0, _RW)], idx_v.at[l], isem)
        for l in range(_L)
    ]
    for c in icopies:
        c.wait()

    def start_gather(g):
        b = g % _NBUF
        return pltpu.async_copy(
            table_hbm.at[idx_v.at[g]], bufs.at[b], gsems.at[b])

    def start_store(g):
        b = g % _NBUF
        return pltpu.async_copy(
            bufs.at[b], out_hbm.at[g, pl.ds(row0, _RW)], ssems.at[b])

    gcopies = [None] * _L
    scopies = [None] * _L
    for g in range(min(_NBUF, _L)):
        gcopies[g] = start_gather(g)
    for g in range(_L):
        ng = g - 1 + _NBUF
        if g >= 1 and ng < _L:
            scopies[g - 1].wait()
            gcopies[ng] = start_gather(ng)
        gcopies[g].wait()
        scopies[g] = start_store(g)
    for g in range(max(_L - _NBUF, 0), _L):
        if scopies[g] is not None:
            scopies[g].wait()


def kernel(token_id, weight):
    out_lbd = _gather(token_id.T, weight)
    return jnp.transpose(out_lbd, (1, 0, 2))
